# baseline (speedup 1.0000x reference)
import functools

import jax
import jax.numpy as jnp
from jax import lax
from jax.experimental import pallas as pl
from jax.experimental.pallas import tpu as pltpu
from jax.experimental.pallas import tpu_sc as plsc

N = 10000
E = 320000
D_FEAT = 128
D_EDGE = 16
D_H = 128
STEPS = 4

NC = 2
NS = 16
NW = NC * NS
CH = 128
CPW = 80
EPAD = NW * CPW * CH
NPAD = 10112
RPS = NPAD // NS
MGC = 4
BIPAD = NW * MGC * CH

_f32 = jnp.float32
_bf16 = jnp.bfloat16


def _table_body(nf, w, b, o):
    o[...] = jnp.dot(nf[...], w[...], preferred_element_type=_f32) + b[...]


def _efw_body(ef8, wbd, o):
    o[...] = jnp.dot(ef8[...], wbd[...], preferred_element_type=_f32).astype(_bf16)


def _gmm_body(a0, a1, w, b, o):
    agg = a0[...] + a1[...]
    o[...] = jnp.dot(agg, w[...], preferred_element_type=_f32) + b[...]


def _final_body(nf, a0, a1, mg, wot, wob, bo, w1t, w1b, b1r, w2, b2r,
                w3, b3r, w4r, b4s, o):
    agg = a0[...] + a1[...]
    nh = jnp.maximum(
        jnp.dot(nf[...], wot[...], preferred_element_type=_f32)
        + jnp.dot(agg, wob[...], preferred_element_type=_f32) + bo[...], 0.0)
    x = jnp.maximum(
        jnp.dot(nh, w1t[...], preferred_element_type=_f32)
        + jnp.dot(mg[...], w1b[...], preferred_element_type=_f32) + b1r[...], 0.0)
    x = jnp.maximum(jnp.dot(x, w2[...], preferred_element_type=_f32) + b2r[...], 0.0)
    x = jnp.maximum(jnp.dot(x, w3[...], preferred_element_type=_f32) + b3r[...], 0.0)
    s = jnp.sum(x * w4r[...], axis=1, keepdims=True) + b4s[0]
    o[...] = jnp.broadcast_to(jax.nn.sigmoid(s), o.shape)


def _edge_pass_body(stage_a, *refs):
    if stage_a:
        (sd2, tab, lin, bi2, mtab, part, h0, mg,
         sdv, gbuf, lb, acc, gsem, ssem) = refs
    else:
        (sd2, tab, lin, part,
         sdv, gbuf, lb, acc, gsem, ssem) = refs
        bi2 = mtab = h0 = mg = None

    cid = lax.axis_index("c")
    sid = lax.axis_index("s")
    wid = sid * NC + cid

    zv = jnp.zeros((16,), _f32)

    @plsc.parallel_loop(0, CH)
    def _zrow(i):
        for j in range(8):
            gbuf[0, i, pl.ds(16 * j, 16)] = zv

    row0 = sid * RPS
    for k in range(RPS // CH):
        pltpu.sync_copy(gbuf.at[0], acc.at[pl.ds(row0 + k * CH, CH)])
    rem = RPS % CH
    if rem:
        pltpu.sync_copy(gbuf.at[0, pl.ds(0, rem)],
                        acc.at[pl.ds(row0 + (RPS // CH) * CH, rem)])
    plsc.subcore_barrier()

    if stage_a:
        pltpu.sync_copy(bi2.at[pl.ds((wid // 2) * 8, 8)], sdv)
        for t in range(MGC):
            r = (wid % 2) * MGC + t
            pltpu.async_copy(mtab.at[sdv.at[r]], gbuf.at[0], gsem.at[0]).wait()
            pltpu.sync_copy(gbuf.at[0], mg.at[pl.ds((wid * MGC + t) * CH, CH)])

    def _stage_idx(blk):
        pltpu.sync_copy(sd2.at[pl.ds(2 * wid * CPW + blk * 8, 8)], sdv)

    def _srow(v):
        return 2 * (v % 4)

    def _drow(v):
        return 2 * (v % 4) + 1

    def _chunk(u, _):
        b = u % 2

        @pl.when(u >= 1)
        def _():
            pltpu.make_async_copy(
                gbuf.at[1 - b], acc.at[sdv.at[_drow(u - 1)]],
                ssem.at[1 - b]).wait()

        @pl.when(u % 4 == 0)
        def _():
            _stage_idx(u // 4)
            pltpu.async_copy(tab.at[sdv.at[0]], gbuf.at[b], gsem.at[b])

        @pl.when(jnp.logical_and(u + 1 < CPW, (u + 1) % 4 != 0))
        def _():
            pltpu.async_copy(tab.at[sdv.at[_srow(u + 1)]], gbuf.at[1 - b],
                             gsem.at[1 - b])

        ebase = (wid * CPW + u) * CH
        pltpu.sync_copy(lin.at[pl.ds(ebase, CH)], lb)
        pltpu.make_async_copy(tab.at[sdv.at[_srow(u)]], gbuf.at[b],
                              gsem.at[b]).wait()

        @plsc.parallel_loop(0, CH)
        def _crow(i):
            for k in range(4):
                w = lb[i, pl.ds(16 * k, 16)]
                a = lax.bitcast_convert_type(w << 16, _f32)
                b2 = lax.bitcast_convert_type(w & jnp.int32(-65536), _f32)
                sa = pl.ds(32 * k, 16)
                sb = pl.ds(32 * k + 16, 16)
                ra = jnp.maximum(gbuf[b, i, sa] + a, 0.0)
                rb = jnp.maximum(gbuf[b, i, sb] + b2, 0.0)
                gbuf[b, i, sa] = ra
                gbuf[b, i, sb] = rb
                if stage_a:
                    ai = lax.bitcast_convert_type(ra, jnp.int32)
                    bi = lax.bitcast_convert_type(rb, jnp.int32)
                    lo = lax.shift_right_logical(ai + 0x8000, 16)
                    hi = (bi + 0x8000) & jnp.int32(-65536)
                    lb[i, pl.ds(16 * k, 16)] = lo | hi

        if stage_a:
            pltpu.sync_copy(lb, h0.at[pl.ds(ebase, CH)])
        pltpu.async_copy(gbuf.at[b], acc.at[sdv.at[_drow(u)]], ssem.at[b],
                         add=True)
        return 0
    lax.fori_loop(0, CPW, _chunk, 0)

    pltpu.make_async_copy(
        gbuf.at[(CPW - 1) % 2], acc.at[sdv.at[_drow(CPW - 1)]],
        ssem.at[(CPW - 1) % 2]).wait()
    plsc.subcore_barrier()
    for k in range(RPS // CH):
        pltpu.sync_copy(acc.at[pl.ds(row0 + k * CH, CH)],
                        part.at[cid, pl.ds(row0 + k * CH, CH)])
    if rem:
        pltpu.sync_copy(acc.at[pl.ds(row0 + (RPS // CH) * CH, rem)],
                        part.at[cid, pl.ds(row0 + (RPS // CH) * CH, rem)])


def _make_edge_pass(stage_a):
    mesh = plsc.VectorSubcoreMesh(core_axis_name="c", subcore_axis_name="s")
    out_type = [jax.ShapeDtypeStruct((NC, NPAD, D_H), _f32)]
    if stage_a:
        out_type += [jax.ShapeDtypeStruct((EPAD, D_H // 2), jnp.int32),
                     jax.ShapeDtypeStruct((BIPAD, D_H), _f32)]
    scratch = [
        pltpu.VMEM((8, CH), jnp.int32),
        pltpu.VMEM((2, CH, D_H), _f32),
        pltpu.VMEM((CH, D_H // 2), jnp.int32),
        pltpu.VMEM_SHARED((NPAD, D_H), _f32),
        pltpu.SemaphoreType.DMA((2,)),
        pltpu.SemaphoreType.DMA((2,)),
    ]
    return pl.kernel(
        functools.partial(_edge_pass_body, stage_a),
        out_type=tuple(out_type),
        mesh=mesh,
        scratch_types=tuple(scratch),
    )


_edge_pass_a = _make_edge_pass(True)
_edge_pass_b = _make_edge_pass(False)


def kernel(mol_a_reprs, node_features, edge_features, edge_index, batch_indices,
           W_in, b_in, W_msg, b_msg, W_out, b_out,
           W1, b1, W2, b2, W3, b3, W4, b4):
    src = edge_index[0]
    dst = edge_index[1]
    src2 = jnp.pad(src, (0, EPAD - E)).reshape(EPAD // CH, CH)
    dst2 = jnp.pad(dst, (0, EPAD - E), constant_values=N).reshape(EPAD // CH, CH)
    sd2 = jnp.stack([src2, dst2], axis=1).reshape(2 * EPAD // CH, CH)
    bi2 = jnp.pad(batch_indices, (0, BIPAD - N)).reshape(BIPAD // CH, CH)
    ef8 = jnp.pad(edge_features, ((0, EPAD - E), (0, 0))).reshape(EPAD // 8, 8 * D_EDGE)

    wbd = jnp.zeros((8 * D_EDGE, 8 * D_H), _f32)
    w_bot = W_in[D_FEAT:]
    for j in range(8):
        wbd = wbd.at[j * D_EDGE:(j + 1) * D_EDGE, j * D_H:(j + 1) * D_H].set(w_bot)
    q = jnp.arange(8 * D_H)
    perm = (q // 128) * 128 + 32 * ((q % 128) // 32) + (q % 32) // 2 + 16 * (q % 2)
    wbd = wbd[:, perm]

    p_tab = pl.pallas_call(
        _table_body,
        out_shape=jax.ShapeDtypeStruct((N, D_H), _f32),
    )(node_features, W_in[:D_FEAT], b_in.reshape(1, D_H))

    RB = 640
    nrb = (EPAD // 8) // RB
    efw8 = pl.pallas_call(
        _efw_body,
        grid=(nrb,),
        in_specs=[
            pl.BlockSpec((RB, 8 * D_EDGE), lambda i: (i, 0)),
            pl.BlockSpec((8 * D_EDGE, 8 * D_H), lambda i: (0, 0)),
        ],
        out_specs=pl.BlockSpec((RB, 8 * D_H), lambda i: (i, 0)),
        out_shape=jax.ShapeDtypeStruct((EPAD // 8, 8 * D_H), _bf16),
    )(ef8, wbd)
    efw = lax.bitcast_convert_type(
        efw8.reshape(EPAD, D_H // 2, 2), jnp.int32)

    parts, h0, mg = _edge_pass_a(sd2, p_tab, efw, bi2, mol_a_reprs)

    gmm = pl.pallas_call(
        _gmm_body,
        out_shape=jax.ShapeDtypeStruct((NPAD, D_H), _f32),
    )
    b_msg_r = b_msg.reshape(1, D_H)
    for _ in range(STEPS):
        g = gmm(parts[0], parts[1], W_msg, b_msg_r)
        parts = _edge_pass_b(sd2, g, h0)[0]

    FB = 1000
    nfb = N // FB
    row_spec = lambda shp: pl.BlockSpec((FB, shp), lambda i: (i, 0))
    full = lambda r, c: pl.BlockSpec((r, c), lambda i: (0, 0))
    out2d = pl.pallas_call(
        _final_body,
        grid=(nfb,),
        in_specs=[
            row_spec(D_FEAT), row_spec(D_H), row_spec(D_H), row_spec(D_H),
            full(D_FEAT, D_H), full(D_H, D_H), full(1, D_H),
            full(D_H, 256), full(D_H, 256), full(1, 256),
            full(256, 128), full(1, 128),
            full(128, 64), full(1, 64),
            full(1, 64),
            pl.BlockSpec(memory_space=pltpu.SMEM),
        ],
        out_specs=row_spec(D_H),
        out_shape=jax.ShapeDtypeStruct((N, D_H), _f32),
    )(node_features, parts[0], parts[1], mg,
      W_out[:D_FEAT], W_out[D_FEAT:], b_out.reshape(1, D_H),
      W1[:D_H], W1[D_H:], b1.reshape(1, 256),
      W2, b2.reshape(1, 128),
      W3, b3.reshape(1, 64),
      W4[:, 0].reshape(1, 64), b4)
    return out2d[:, 0]


# speedup vs baseline: 1.0080x; 1.0080x over previous
import functools

import jax
import jax.numpy as jnp
from jax import lax
from jax.experimental import pallas as pl
from jax.experimental.pallas import tpu as pltpu
from jax.experimental.pallas import tpu_sc as plsc

N = 10000
E = 320000
D_FEAT = 128
D_EDGE = 16
D_H = 128
STEPS = 4

NC = 2
NS = 16
NW = NC * NS
CH = 128
CPW = 80
EPAD = NW * CPW * CH
NPAD = 10112
RPS = NPAD // NS
MGC = 4
BIPAD = NW * MGC * CH

_f32 = jnp.float32
_bf16 = jnp.bfloat16


def _table_body(nf, w, b, o):
    o[...] = jnp.dot(nf[...], w[...], preferred_element_type=_f32) + b[...]


def _efw_body(ef8, wbd, o):
    o[...] = jnp.dot(ef8[...], wbd[...], preferred_element_type=_f32).astype(_bf16)


def _gmm_body(a0, a1, w, b, o):
    agg = a0[...] + a1[...]
    o[...] = jnp.dot(agg, w[...], preferred_element_type=_f32) + b[...]


def _final_body(nf, a0, a1, mg, wot, wob, bo, w1t, w1b, b1r, w2, b2r,
                w3, b3r, w4r, b4s, o):
    agg = a0[...] + a1[...]
    nh = jnp.maximum(
        jnp.dot(nf[...], wot[...], preferred_element_type=_f32)
        + jnp.dot(agg, wob[...], preferred_element_type=_f32) + bo[...], 0.0)
    x = jnp.maximum(
        jnp.dot(nh, w1t[...], preferred_element_type=_f32)
        + jnp.dot(mg[...], w1b[...], preferred_element_type=_f32) + b1r[...], 0.0)
    x = jnp.maximum(jnp.dot(x, w2[...], preferred_element_type=_f32) + b2r[...], 0.0)
    x = jnp.maximum(jnp.dot(x, w3[...], preferred_element_type=_f32) + b3r[...], 0.0)
    s = jnp.sum(x * w4r[...], axis=1, keepdims=True) + b4s[0]
    o[...] = jnp.broadcast_to(jax.nn.sigmoid(s), o.shape)


def _edge_pass_body(stage_a, *refs):
    if stage_a:
        (sd2, tab, lin, bi2, mtab, part, h0, mg,
         sdv, gb0, gb1, lb, acc, gsem0, gsem1) = refs
    else:
        (sd2, tab, lin, part,
         sdv, gb0, gb1, lb, acc, gsem0, gsem1) = refs
        bi2 = mtab = h0 = mg = None

    cid = lax.axis_index("c")
    sid = lax.axis_index("s")
    wid = sid * NC + cid

    zv = jnp.zeros((16,), _f32)

    def _zrow(i, _):
        for j in range(8):
            gb0[i, pl.ds(16 * j, 16)] = zv
        return 0
    lax.fori_loop(0, CH, _zrow, 0)

    row0 = sid * RPS
    for k in range(RPS // CH):
        pltpu.sync_copy(gb0, acc.at[pl.ds(row0 + k * CH, CH)])
    rem = RPS % CH
    if rem:
        pltpu.sync_copy(gb0.at[pl.ds(0, rem)],
                        acc.at[pl.ds(row0 + (RPS // CH) * CH, rem)])
    plsc.subcore_barrier()

    if stage_a:
        pltpu.sync_copy(bi2.at[pl.ds((wid // 2) * 8, 8)], sdv)
        for t in range(MGC):
            r = (wid % 2) * MGC + t
            pltpu.async_copy(mtab.at[sdv.at[r]], gb0, gsem0).wait()
            pltpu.sync_copy(gb0, mg.at[pl.ds((wid * MGC + t) * CH, CH)])

    gbufs = (gb0, gb1)
    gsems = (gsem0, gsem1)

    def _compute(gb):
        def _crow(i, _):
            for k in range(4):
                w = lb[i, pl.ds(16 * k, 16)]
                a = lax.bitcast_convert_type(w << 16, _f32)
                b2 = lax.bitcast_convert_type(w & jnp.int32(-65536), _f32)
                sa = pl.ds(32 * k, 16)
                sb = pl.ds(32 * k + 16, 16)
                ra = jnp.maximum(gb[i, sa] + a, 0.0)
                rb = jnp.maximum(gb[i, sb] + b2, 0.0)
                gb[i, sa] = ra
                gb[i, sb] = rb
                if stage_a:
                    ai = lax.bitcast_convert_type(ra, jnp.int32)
                    bi = lax.bitcast_convert_type(rb, jnp.int32)
                    lo = lax.shift_right_logical(ai + 0x8000, 16)
                    hi = (bi + 0x8000) & jnp.int32(-65536)
                    lb[i, pl.ds(16 * k, 16)] = lo | hi
            return 0
        lax.fori_loop(0, CH, _crow, 0)

    def _blk(p, _):
        pltpu.sync_copy(sd2.at[pl.ds(2 * wid * CPW + p * 8, 8)], sdv)
        cps = [None] * 5
        cps[0] = pltpu.async_copy(tab.at[sdv.at[0]], gb0, gsem0)
        for u2 in range(4):
            gb = gbufs[u2 % 2]
            if u2 < 3:
                cps[u2 + 1] = pltpu.async_copy(
                    tab.at[sdv.at[2 * (u2 + 1)]], gbufs[(u2 + 1) % 2],
                    gsems[(u2 + 1) % 2])
            ebase = (wid * CPW + p * 4 + u2) * CH
            pltpu.sync_copy(lin.at[pl.ds(ebase, CH)], lb)
            cps[u2].wait()
            _compute(gb)
            if stage_a:
                pltpu.sync_copy(lb, h0.at[pl.ds(ebase, CH)])
            pltpu.sync_copy(gb, acc.at[sdv.at[2 * u2 + 1]], add=True)
        return 0
    lax.fori_loop(0, CPW // 4, _blk, 0)

    plsc.subcore_barrier()
    for k in range(RPS // CH):
        pltpu.sync_copy(acc.at[pl.ds(row0 + k * CH, CH)],
                        part.at[cid, pl.ds(row0 + k * CH, CH)])
    if rem:
        pltpu.sync_copy(acc.at[pl.ds(row0 + (RPS // CH) * CH, rem)],
                        part.at[cid, pl.ds(row0 + (RPS // CH) * CH, rem)])


def _make_edge_pass(stage_a):
    mesh = plsc.VectorSubcoreMesh(core_axis_name="c", subcore_axis_name="s")
    out_type = [jax.ShapeDtypeStruct((NC, NPAD, D_H), _f32)]
    if stage_a:
        out_type += [jax.ShapeDtypeStruct((EPAD, D_H // 2), jnp.int32),
                     jax.ShapeDtypeStruct((BIPAD, D_H), _f32)]
    scratch = [
        pltpu.VMEM((8, CH), jnp.int32),
        pltpu.VMEM((CH, D_H), _f32),
        pltpu.VMEM((CH, D_H), _f32),
        pltpu.VMEM((CH, D_H // 2), jnp.int32),
        pltpu.VMEM_SHARED((NPAD, D_H), _f32),
        pltpu.SemaphoreType.DMA,
        pltpu.SemaphoreType.DMA,
    ]
    return pl.kernel(
        functools.partial(_edge_pass_body, stage_a),
        out_type=tuple(out_type),
        mesh=mesh,
        scratch_types=tuple(scratch),
    )


_edge_pass_a = _make_edge_pass(True)
_edge_pass_b = _make_edge_pass(False)


def kernel(mol_a_reprs, node_features, edge_features, edge_index, batch_indices,
           W_in, b_in, W_msg, b_msg, W_out, b_out,
           W1, b1, W2, b2, W3, b3, W4, b4):
    src = edge_index[0]
    dst = edge_index[1]
    src2 = jnp.pad(src, (0, EPAD - E)).reshape(EPAD // CH, CH)
    dst2 = jnp.pad(dst, (0, EPAD - E), constant_values=N).reshape(EPAD // CH, CH)
    sd2 = jnp.stack([src2, dst2], axis=1).reshape(2 * EPAD // CH, CH)
    bi2 = jnp.pad(batch_indices, (0, BIPAD - N)).reshape(BIPAD // CH, CH)
    ef8 = jnp.pad(edge_features, ((0, EPAD - E), (0, 0))).reshape(EPAD // 8, 8 * D_EDGE)

    wbd = jnp.zeros((8 * D_EDGE, 8 * D_H), _f32)
    w_bot = W_in[D_FEAT:]
    for j in range(8):
        wbd = wbd.at[j * D_EDGE:(j + 1) * D_EDGE, j * D_H:(j + 1) * D_H].set(w_bot)
    q = jnp.arange(8 * D_H)
    perm = (q // 128) * 128 + 32 * ((q % 128) // 32) + (q % 32) // 2 + 16 * (q % 2)
    wbd = wbd[:, perm]

    p_tab = pl.pallas_call(
        _table_body,
        out_shape=jax.ShapeDtypeStruct((N, D_H), _f32),
    )(node_features, W_in[:D_FEAT], b_in.reshape(1, D_H))

    RB = 640
    nrb = (EPAD // 8) // RB
    efw8 = pl.pallas_call(
        _efw_body,
        grid=(nrb,),
        in_specs=[
            pl.BlockSpec((RB, 8 * D_EDGE), lambda i: (i, 0)),
            pl.BlockSpec((8 * D_EDGE, 8 * D_H), lambda i: (0, 0)),
        ],
        out_specs=pl.BlockSpec((RB, 8 * D_H), lambda i: (i, 0)),
        out_shape=jax.ShapeDtypeStruct((EPAD // 8, 8 * D_H), _bf16),
    )(ef8, wbd)
    efw = lax.bitcast_convert_type(
        efw8.reshape(EPAD, D_H // 2, 2), jnp.int32)

    parts, h0, mg = _edge_pass_a(sd2, p_tab, efw, bi2, mol_a_reprs)

    gmm = pl.pallas_call(
        _gmm_body,
        out_shape=jax.ShapeDtypeStruct((NPAD, D_H), _f32),
    )
    b_msg_r = b_msg.reshape(1, D_H)
    for _ in range(STEPS):
        g = gmm(parts[0], parts[1], W_msg, b_msg_r)
        parts = _edge_pass_b(sd2, g, h0)[0]

    FB = 1000
    nfb = N // FB
    row_spec = lambda shp: pl.BlockSpec((FB, shp), lambda i: (i, 0))
    full = lambda r, c: pl.BlockSpec((r, c), lambda i: (0, 0))
    out2d = pl.pallas_call(
        _final_body,
        grid=(nfb,),
        in_specs=[
            row_spec(D_FEAT), row_spec(D_H), row_spec(D_H), row_spec(D_H),
            full(D_FEAT, D_H), full(D_H, D_H), full(1, D_H),
            full(D_H, 256), full(D_H, 256), full(1, 256),
            full(256, 128), full(1, 128),
            full(128, 64), full(1, 64),
            full(1, 64),
            pl.BlockSpec(memory_space=pltpu.SMEM),
        ],
        out_specs=row_spec(D_H),
        out_shape=jax.ShapeDtypeStruct((N, D_H), _f32),
    )(node_features, parts[0], parts[1], mg,
      W_out[:D_FEAT], W_out[D_FEAT:], b_out.reshape(1, D_H),
      W1[:D_H], W1[D_H:], b1.reshape(1, 256),
      W2, b2.reshape(1, 128),
      W3, b3.reshape(1, 64),
      W4[:, 0].reshape(1, 64), b4)
    return out2d[:, 0]


# speedup vs baseline: 4.8361x; 4.7975x over previous
import functools

import jax
import jax.numpy as jnp
from jax import lax
from jax.experimental import pallas as pl
from jax.experimental.pallas import tpu as pltpu
from jax.experimental.pallas import tpu_sc as plsc

N = 10000
E = 320000
D_FEAT = 128
D_EDGE = 16
D_H = 128
STEPS = 4

NC = 2
NS = 16
NW = NC * NS
CH = 64
CPW = 160
EPAD = NW * CPW * CH
NPAD = 10112
RPS = NPAD // NS
MGC = 8
BIPAD = NW * MGC * CH

_f32 = jnp.float32


def _table_body(nf, w, b, o):
    o[...] = jnp.dot(nf[...], w[...], preferred_element_type=_f32) + b[...]


def _efw_body(ef8, wbd, o):
    o[...] = jnp.dot(ef8[...], wbd[...], preferred_element_type=_f32)


def _gmm_body(a0, a1, w, b, o):
    agg = a0[...] + a1[...]
    o[...] = jnp.dot(agg, w[...], preferred_element_type=_f32) + b[...]


def _final_body(nf, a0, a1, mg, wot, wob, bo, w1t, w1b, b1r, w2, b2r,
                w3, b3r, w4r, b4s, o):
    agg = a0[...] + a1[...]
    nh = jnp.maximum(
        jnp.dot(nf[...], wot[...], preferred_element_type=_f32)
        + jnp.dot(agg, wob[...], preferred_element_type=_f32) + bo[...], 0.0)
    x = jnp.maximum(
        jnp.dot(nh, w1t[...], preferred_element_type=_f32)
        + jnp.dot(mg[...], w1b[...], preferred_element_type=_f32) + b1r[...], 0.0)
    x = jnp.maximum(jnp.dot(x, w2[...], preferred_element_type=_f32) + b2r[...], 0.0)
    x = jnp.maximum(jnp.dot(x, w3[...], preferred_element_type=_f32) + b3r[...], 0.0)
    s = jnp.sum(x * w4r[...], axis=1, keepdims=True) + b4s[0]
    o[...] = jnp.broadcast_to(jax.nn.sigmoid(s), o.shape)


def _edge_pass_body(stage_a, *refs):
    if stage_a:
        (sd2, tab, lin, bi2, mtab, part, h0, mg,
         sdv, gb0, gb1, lb, acc, gsem0, gsem1) = refs
    else:
        (sd2, tab, lin, part,
         sdv, gb0, gb1, lb, acc, gsem0, gsem1) = refs
        bi2 = mtab = h0 = mg = None

    cid = lax.axis_index("c")
    sid = lax.axis_index("s")
    wid = sid * NC + cid

    zv = jnp.zeros((16,), _f32)

    def _zrow(i, _):
        for j in range(8):
            lb[i, pl.ds(16 * j, 16)] = zv
        return 0
    lax.fori_loop(0, CH, _zrow, 0)

    row0 = sid * RPS
    nz = RPS // CH
    for k in range(nz):
        pltpu.sync_copy(lb, acc.at[pl.ds(row0 + k * CH, CH)])
    rem = RPS % CH
    if rem:
        pltpu.sync_copy(lb.at[pl.ds(0, rem)],
                        acc.at[pl.ds(row0 + nz * CH, rem)])
    plsc.subcore_barrier()

    if stage_a:
        pltpu.sync_copy(bi2.at[pl.ds(wid * MGC, MGC)], sdv)
        for t in range(MGC):
            pltpu.async_copy(mtab.at[sdv.at[t]], gb0, gsem0).wait()
            pltpu.sync_copy(gb0, mg.at[pl.ds((wid * MGC + t) * CH, CH)])

    gbufs = (gb0, gb1)
    gsems = (gsem0, gsem1)

    def _compute(gb):
        def _crow(i, _):
            for j in range(8):
                sl = pl.ds(16 * j, 16)
                gb[i, sl] = jnp.maximum(gb[i, sl] + lb[i, sl], 0.0)
            return 0
        lax.fori_loop(0, CH, _crow, 0)

    def _blk(p, _):
        pltpu.sync_copy(sd2.at[pl.ds(2 * wid * CPW + p * 8, 8)], sdv)
        cps = [None] * 5
        cps[0] = pltpu.async_copy(tab.at[sdv.at[0]], gb0, gsem0)
        for u2 in range(4):
            gb = gbufs[u2 % 2]
            if u2 < 3:
                cps[u2 + 1] = pltpu.async_copy(
                    tab.at[sdv.at[2 * (u2 + 1)]], gbufs[(u2 + 1) % 2],
                    gsems[(u2 + 1) % 2])
            ebase = (wid * CPW + p * 4 + u2) * CH
            pltpu.sync_copy(lin.at[pl.ds(ebase, CH)], lb)
            cps[u2].wait()
            _compute(gb)
            if stage_a:
                pltpu.sync_copy(gb, h0.at[pl.ds(ebase, CH)])
            pltpu.sync_copy(gb, acc.at[sdv.at[2 * u2 + 1]], add=True)
        return 0
    lax.fori_loop(0, CPW // 4, _blk, 0)

    plsc.subcore_barrier()
    for k in range(nz):
        pltpu.sync_copy(acc.at[pl.ds(row0 + k * CH, CH)],
                        part.at[cid, pl.ds(row0 + k * CH, CH)])
    if rem:
        pltpu.sync_copy(acc.at[pl.ds(row0 + nz * CH, rem)],
                        part.at[cid, pl.ds(row0 + nz * CH, rem)])


def _make_edge_pass(stage_a):
    mesh = plsc.VectorSubcoreMesh(core_axis_name="c", subcore_axis_name="s")
    out_type = [jax.ShapeDtypeStruct((NC, NPAD, D_H), _f32)]
    if stage_a:
        out_type += [jax.ShapeDtypeStruct((EPAD, D_H), _f32),
                     jax.ShapeDtypeStruct((BIPAD, D_H), _f32)]
    scratch = [
        pltpu.VMEM((8, CH), jnp.int32),
        pltpu.VMEM((CH, D_H), _f32),
        pltpu.VMEM((CH, D_H), _f32),
        pltpu.VMEM((CH, D_H), _f32),
        pltpu.VMEM_SHARED((NPAD, D_H), _f32),
        pltpu.SemaphoreType.DMA,
        pltpu.SemaphoreType.DMA,
    ]
    return pl.kernel(
        functools.partial(_edge_pass_body, stage_a),
        out_type=tuple(out_type),
        mesh=mesh,
        scratch_types=tuple(scratch),
    )


_edge_pass_a = _make_edge_pass(True)
_edge_pass_b = _make_edge_pass(False)


def kernel(mol_a_reprs, node_features, edge_features, edge_index, batch_indices,
           W_in, b_in, W_msg, b_msg, W_out, b_out,
           W1, b1, W2, b2, W3, b3, W4, b4):
    src = edge_index[0]
    dst = edge_index[1]
    src2 = jnp.pad(src, (0, EPAD - E)).reshape(EPAD // CH, CH)
    dst2 = jnp.pad(dst, (0, EPAD - E), constant_values=N).reshape(EPAD // CH, CH)
    sd2 = jnp.stack([src2, dst2], axis=1).reshape(2 * EPAD // CH, CH)
    bi2 = jnp.pad(batch_indices, (0, BIPAD - N)).reshape(BIPAD // CH, CH)
    ef8 = jnp.pad(edge_features, ((0, EPAD - E), (0, 0))).reshape(EPAD // 8, 8 * D_EDGE)

    wbd = jnp.zeros((8 * D_EDGE, 8 * D_H), _f32)
    w_bot = W_in[D_FEAT:]
    for j in range(8):
        wbd = wbd.at[j * D_EDGE:(j + 1) * D_EDGE, j * D_H:(j + 1) * D_H].set(w_bot)

    p_tab = pl.pallas_call(
        _table_body,
        out_shape=jax.ShapeDtypeStruct((N, D_H), _f32),
    )(node_features, W_in[:D_FEAT], b_in.reshape(1, D_H))

    RB = 640
    nrb = (EPAD // 8) // RB
    efw8 = pl.pallas_call(
        _efw_body,
        grid=(nrb,),
        in_specs=[
            pl.BlockSpec((RB, 8 * D_EDGE), lambda i: (i, 0)),
            pl.BlockSpec((8 * D_EDGE, 8 * D_H), lambda i: (0, 0)),
        ],
        out_specs=pl.BlockSpec((RB, 8 * D_H), lambda i: (i, 0)),
        out_shape=jax.ShapeDtypeStruct((EPAD // 8, 8 * D_H), _f32),
    )(ef8, wbd)
    efw = efw8.reshape(EPAD, D_H)

    parts, h0, mg = _edge_pass_a(sd2, p_tab, efw, bi2, mol_a_reprs)

    gmm = pl.pallas_call(
        _gmm_body,
        out_shape=jax.ShapeDtypeStruct((NPAD, D_H), _f32),
    )
    b_msg_r = b_msg.reshape(1, D_H)
    for _ in range(STEPS):
        g = gmm(parts[0], parts[1], W_msg, b_msg_r)
        parts = _edge_pass_b(sd2, g, h0)[0]

    FB = 1000
    nfb = N // FB
    row_spec = lambda shp: pl.BlockSpec((FB, shp), lambda i: (i, 0))
    full = lambda r, c: pl.BlockSpec((r, c), lambda i: (0, 0))
    out2d = pl.pallas_call(
        _final_body,
        grid=(nfb,),
        in_specs=[
            row_spec(D_FEAT), row_spec(D_H), row_spec(D_H), row_spec(D_H),
            full(D_FEAT, D_H), full(D_H, D_H), full(1, D_H),
            full(D_H, 256), full(D_H, 256), full(1, 256),
            full(256, 128), full(1, 128),
            full(128, 64), full(1, 64),
            full(1, 64),
            pl.BlockSpec(memory_space=pltpu.SMEM),
        ],
        out_specs=row_spec(D_H),
        out_shape=jax.ShapeDtypeStruct((N, D_H), _f32),
    )(node_features, parts[0], parts[1], mg,
      W_out[:D_FEAT], W_out[D_FEAT:], b_out.reshape(1, D_H),
      W1[:D_H], W1[D_H:], b1.reshape(1, 256),
      W2, b2.reshape(1, 128),
      W3, b3.reshape(1, 64),
      W4[:, 0].reshape(1, 64), b4)
    return out2d[:, 0]


# speedup vs baseline: 5.3112x; 1.0982x over previous
import functools

import jax
import jax.numpy as jnp
from jax import lax
from jax.experimental import pallas as pl
from jax.experimental.pallas import tpu as pltpu
from jax.experimental.pallas import tpu_sc as plsc

N = 10000
E = 320000
D_FEAT = 128
D_EDGE = 16
D_H = 128
STEPS = 4

NC = 2
NS = 16
NW = NC * NS
CH = 64
CPW = 160
CPW0 = 196
CPW1 = 124
EPAD = NW * CPW * CH
NPAD = 10112
RPS = NPAD // NS
MGC = 8
BIPAD = NW * MGC * CH

_f32 = jnp.float32


def _table_body(nf, w, b, o):
    o[...] = jnp.dot(nf[...], w[...], preferred_element_type=_f32) + b[...]


def _efw_body(ef8, wbd, o):
    o[...] = jnp.dot(ef8[...], wbd[...], preferred_element_type=_f32)


def _gmm_body(a0, a1, w, b, o):
    agg = a0[...] + a1[...]
    o[...] = jnp.dot(agg, w[...], preferred_element_type=_f32) + b[...]


def _final_body(nf, a0, a1, mg, wot, wob, bo, w1t, w1b, b1r, w2, b2r,
                w3, b3r, w4r, b4s, o):
    agg = a0[...] + a1[...]
    nh = jnp.maximum(
        jnp.dot(nf[...], wot[...], preferred_element_type=_f32)
        + jnp.dot(agg, wob[...], preferred_element_type=_f32) + bo[...], 0.0)
    x = jnp.maximum(
        jnp.dot(nh, w1t[...], preferred_element_type=_f32)
        + jnp.dot(mg[...], w1b[...], preferred_element_type=_f32) + b1r[...], 0.0)
    x = jnp.maximum(jnp.dot(x, w2[...], preferred_element_type=_f32) + b2r[...], 0.0)
    x = jnp.maximum(jnp.dot(x, w3[...], preferred_element_type=_f32) + b3r[...], 0.0)
    s = jnp.sum(x * w4r[...], axis=1, keepdims=True) + b4s[0]
    o[...] = jnp.broadcast_to(jax.nn.sigmoid(s), o.shape)


def _edge_pass_body(stage_a, *refs):
    if stage_a:
        (sd2, tab, lin, bi2, mtab, part, h0, mg,
         sdv, gb0, gb1, lb, acc, gsem0, gsem1) = refs
    else:
        (sd2, tab, lin, part,
         sdv, gb0, gb1, lb, acc, gsem0, gsem1) = refs
        bi2 = mtab = h0 = mg = None

    cid = lax.axis_index("c")
    sid = lax.axis_index("s")
    wid = sid * NC + cid

    zv = jnp.zeros((16,), _f32)

    def _zrow(i, _):
        for j in range(8):
            lb[i, pl.ds(16 * j, 16)] = zv
        return 0
    lax.fori_loop(0, CH, _zrow, 0)

    row0 = sid * RPS
    nz = RPS // CH
    for k in range(nz):
        pltpu.sync_copy(lb, acc.at[pl.ds(row0 + k * CH, CH)])
    rem = RPS % CH
    if rem:
        pltpu.sync_copy(lb.at[pl.ds(0, rem)],
                        acc.at[pl.ds(row0 + nz * CH, rem)])
    plsc.subcore_barrier()

    if stage_a:
        pltpu.sync_copy(bi2.at[pl.ds(wid * MGC, MGC)], sdv)
        for t in range(MGC):
            pltpu.async_copy(mtab.at[sdv.at[t]], gb0, gsem0).wait()
            pltpu.sync_copy(gb0, mg.at[pl.ds((wid * MGC + t) * CH, CH)])

    gbufs = (gb0, gb1)
    gsems = (gsem0, gsem1)

    cbase = jnp.where(cid == 0, sid * CPW0, 16 * CPW0 + sid * CPW1)
    nblk = jnp.where(cid == 0, CPW0 // 4, CPW1 // 4)

    def _compute(gb):
        def _crow(i, _):
            for j in range(8):
                sl = pl.ds(16 * j, 16)
                gb[i, sl] = jnp.maximum(gb[i, sl] + lb[i, sl], 0.0)
            return 0
        lax.fori_loop(0, CH, _crow, 0)

    def _blk(p, _):
        pltpu.sync_copy(sd2.at[pl.ds(2 * cbase + p * 8, 8)], sdv)
        cps = [None] * 5
        cps[0] = pltpu.async_copy(tab.at[sdv.at[0]], gb0, gsem0)
        for u2 in range(4):
            gb = gbufs[u2 % 2]
            if u2 < 3:
                cps[u2 + 1] = pltpu.async_copy(
                    tab.at[sdv.at[2 * (u2 + 1)]], gbufs[(u2 + 1) % 2],
                    gsems[(u2 + 1) % 2])
            ebase = (cbase + p * 4 + u2) * CH
            pltpu.sync_copy(lin.at[pl.ds(ebase, CH)], lb)
            cps[u2].wait()
            _compute(gb)
            if stage_a:
                pltpu.sync_copy(gb, h0.at[pl.ds(ebase, CH)])
            pltpu.sync_copy(gb, acc.at[sdv.at[2 * u2 + 1]], add=True)
        return 0
    lax.fori_loop(0, nblk, _blk, 0)

    plsc.subcore_barrier()
    for k in range(nz):
        pltpu.sync_copy(acc.at[pl.ds(row0 + k * CH, CH)],
                        part.at[cid, pl.ds(row0 + k * CH, CH)])
    if rem:
        pltpu.sync_copy(acc.at[pl.ds(row0 + nz * CH, rem)],
                        part.at[cid, pl.ds(row0 + nz * CH, rem)])


def _make_edge_pass(stage_a):
    mesh = plsc.VectorSubcoreMesh(core_axis_name="c", subcore_axis_name="s")
    out_type = [jax.ShapeDtypeStruct((NC, NPAD, D_H), _f32)]
    if stage_a:
        out_type += [jax.ShapeDtypeStruct((EPAD, D_H), _f32),
                     jax.ShapeDtypeStruct((BIPAD, D_H), _f32)]
    scratch = [
        pltpu.VMEM((8, CH), jnp.int32),
        pltpu.VMEM((CH, D_H), _f32),
        pltpu.VMEM((CH, D_H), _f32),
        pltpu.VMEM((CH, D_H), _f32),
        pltpu.VMEM_SHARED((NPAD, D_H), _f32),
        pltpu.SemaphoreType.DMA,
        pltpu.SemaphoreType.DMA,
    ]
    return pl.kernel(
        functools.partial(_edge_pass_body, stage_a),
        out_type=tuple(out_type),
        mesh=mesh,
        scratch_types=tuple(scratch),
    )


_edge_pass_a = _make_edge_pass(True)
_edge_pass_b = _make_edge_pass(False)


def kernel(mol_a_reprs, node_features, edge_features, edge_index, batch_indices,
           W_in, b_in, W_msg, b_msg, W_out, b_out,
           W1, b1, W2, b2, W3, b3, W4, b4):
    src = edge_index[0]
    dst = edge_index[1]
    src2 = jnp.pad(src, (0, EPAD - E)).reshape(EPAD // CH, CH)
    dst2 = jnp.pad(dst, (0, EPAD - E), constant_values=N).reshape(EPAD // CH, CH)
    sd2 = jnp.stack([src2, dst2], axis=1).reshape(2 * EPAD // CH, CH)
    bi2 = jnp.pad(batch_indices, (0, BIPAD - N)).reshape(BIPAD // CH, CH)
    ef8 = jnp.pad(edge_features, ((0, EPAD - E), (0, 0))).reshape(EPAD // 8, 8 * D_EDGE)

    wbd = jnp.zeros((8 * D_EDGE, 8 * D_H), _f32)
    w_bot = W_in[D_FEAT:]
    for j in range(8):
        wbd = wbd.at[j * D_EDGE:(j + 1) * D_EDGE, j * D_H:(j + 1) * D_H].set(w_bot)

    p_tab = pl.pallas_call(
        _table_body,
        out_shape=jax.ShapeDtypeStruct((N, D_H), _f32),
    )(node_features, W_in[:D_FEAT], b_in.reshape(1, D_H))

    RB = 640
    nrb = (EPAD // 8) // RB
    efw8 = pl.pallas_call(
        _efw_body,
        grid=(nrb,),
        in_specs=[
            pl.BlockSpec((RB, 8 * D_EDGE), lambda i: (i, 0)),
            pl.BlockSpec((8 * D_EDGE, 8 * D_H), lambda i: (0, 0)),
        ],
        out_specs=pl.BlockSpec((RB, 8 * D_H), lambda i: (i, 0)),
        out_shape=jax.ShapeDtypeStruct((EPAD // 8, 8 * D_H), _f32),
    )(ef8, wbd)
    efw = efw8.reshape(EPAD, D_H)

    parts, h0, mg = _edge_pass_a(sd2, p_tab, efw, bi2, mol_a_reprs)

    gmm = pl.pallas_call(
        _gmm_body,
        out_shape=jax.ShapeDtypeStruct((NPAD, D_H), _f32),
    )
    b_msg_r = b_msg.reshape(1, D_H)
    for _ in range(STEPS):
        g = gmm(parts[0], parts[1], W_msg, b_msg_r)
        parts = _edge_pass_b(sd2, g, h0)[0]

    FB = 1000
    nfb = N // FB
    row_spec = lambda shp: pl.BlockSpec((FB, shp), lambda i: (i, 0))
    full = lambda r, c: pl.BlockSpec((r, c), lambda i: (0, 0))
    out2d = pl.pallas_call(
        _final_body,
        grid=(nfb,),
        in_specs=[
            row_spec(D_FEAT), row_spec(D_H), row_spec(D_H), row_spec(D_H),
            full(D_FEAT, D_H), full(D_H, D_H), full(1, D_H),
            full(D_H, 256), full(D_H, 256), full(1, 256),
            full(256, 128), full(1, 128),
            full(128, 64), full(1, 64),
            full(1, 64),
            pl.BlockSpec(memory_space=pltpu.SMEM),
        ],
        out_specs=row_spec(D_H),
        out_shape=jax.ShapeDtypeStruct((N, D_H), _f32),
    )(node_features, parts[0], parts[1], mg,
      W_out[:D_FEAT], W_out[D_FEAT:], b_out.reshape(1, D_H),
      W1[:D_H], W1[D_H:], b1.reshape(1, 256),
      W2, b2.reshape(1, 128),
      W3, b3.reshape(1, 64),
      W4[:, 0].reshape(1, 64), b4)
    return out2d[:, 0]


# speedup vs baseline: 5.5795x; 1.0505x over previous
import functools

import jax
import jax.numpy as jnp
from jax import lax
from jax.experimental import pallas as pl
from jax.experimental.pallas import tpu as pltpu
from jax.experimental.pallas import tpu_sc as plsc

N = 10000
E = 320000
D_FEAT = 128
D_EDGE = 16
D_H = 128
STEPS = 4

NC = 2
NS = 16
NW = NC * NS
CH = 64
CPW = 160
CPW0 = 200
CPW1 = 120
EPAD = NW * CPW * CH
NPAD = 10112
RPS = NPAD // NS
MGC = 8
BIPAD = NW * MGC * CH

_f32 = jnp.float32


def _table_body(nf, w, b, o):
    o[...] = jnp.dot(nf[...], w[...], preferred_element_type=_f32) + b[...]


def _efw_body(ef8, wbd, o):
    o[...] = jnp.dot(ef8[...], wbd[...], preferred_element_type=_f32)


def _gmm_body(a0, a1, w, b, o):
    agg = a0[...] + a1[...]
    o[...] = jnp.dot(agg, w[...], preferred_element_type=_f32) + b[...]


def _final_body(nf, a0, a1, mg, wot, wob, bo, w1t, w1b, b1r, w2, b2r,
                w3, b3r, w4r, b4s, o):
    agg = a0[...] + a1[...]
    nh = jnp.maximum(
        jnp.dot(nf[...], wot[...], preferred_element_type=_f32)
        + jnp.dot(agg, wob[...], preferred_element_type=_f32) + bo[...], 0.0)
    x = jnp.maximum(
        jnp.dot(nh, w1t[...], preferred_element_type=_f32)
        + jnp.dot(mg[...], w1b[...], preferred_element_type=_f32) + b1r[...], 0.0)
    x = jnp.maximum(jnp.dot(x, w2[...], preferred_element_type=_f32) + b2r[...], 0.0)
    x = jnp.maximum(jnp.dot(x, w3[...], preferred_element_type=_f32) + b3r[...], 0.0)
    s = jnp.sum(x * w4r[...], axis=1, keepdims=True) + b4s[0]
    o[...] = jnp.broadcast_to(jax.nn.sigmoid(s), o.shape)


def _edge_pass_body(stage_a, *refs):
    if stage_a:
        (sd2, tab, lin, bi2, mtab, part, h0, mg,
         sdv, gb0, gb1, lb0, lb1, acc,
         gsem0, gsem1, lsem0, lsem1, ssem0, ssem1, hsem0, hsem1) = refs
    else:
        (sd2, tab, lin, part,
         sdv, gb0, gb1, lb0, lb1, acc,
         gsem0, gsem1, lsem0, lsem1, ssem0, ssem1, hsem0, hsem1) = refs
        bi2 = mtab = h0 = mg = None

    cid = lax.axis_index("c")
    sid = lax.axis_index("s")
    wid = sid * NC + cid

    zv = jnp.zeros((16,), _f32)

    def _zrow(i, _):
        for j in range(8):
            lb0[i, pl.ds(16 * j, 16)] = zv
        return 0
    lax.fori_loop(0, CH, _zrow, 0)

    row0 = sid * RPS
    nz = RPS // CH
    for k in range(nz):
        pltpu.sync_copy(lb0, acc.at[pl.ds(row0 + k * CH, CH)])
    rem = RPS % CH
    if rem:
        pltpu.sync_copy(lb0.at[pl.ds(0, rem)],
                        acc.at[pl.ds(row0 + nz * CH, rem)])
    plsc.subcore_barrier()

    if stage_a:
        pltpu.sync_copy(bi2.at[pl.ds(wid * MGC, MGC)], sdv.at[pl.ds(0, 8)])
        for t in range(MGC):
            pltpu.async_copy(mtab.at[sdv.at[t]], gb0, gsem0).wait()
            pltpu.sync_copy(gb0, mg.at[pl.ds((wid * MGC + t) * CH, CH)])

    gbufs = (gb0, gb1)
    lbufs = (lb0, lb1)
    gsems = (gsem0, gsem1)
    lsems = (lsem0, lsem1)
    ssems = (ssem0, ssem1)
    hsems = (hsem0, hsem1)

    cbase = jnp.where(cid == 0, sid * CPW0, 16 * CPW0 + sid * CPW1)
    nblk = jnp.where(cid == 0, CPW0 // 8, CPW1 // 8)

    def _compute(gb, lb):
        def _crow(i, _):
            for j in range(8):
                sl = pl.ds(16 * j, 16)
                gb[i, sl] = jnp.maximum(gb[i, sl] + lb[i, sl], 0.0)
            return 0
        lax.fori_loop(0, CH, _crow, 0)

    def _blk(p, _):
        pltpu.sync_copy(sd2.at[pl.ds(2 * cbase + p * 16, 16)], sdv)
        base = cbase + p * 8
        gcp = [None] * 8
        lcp = [None] * 8
        scp = [None] * 8
        hcp = [None] * 8
        for v in range(2):
            gcp[v] = pltpu.async_copy(tab.at[sdv.at[2 * v]], gbufs[v], gsems[v])
            lcp[v] = pltpu.async_copy(lin.at[pl.ds((base + v) * CH, CH)],
                                      lbufs[v], lsems[v])
        for u in range(8):
            b = u % 2
            if u >= 1:
                scp[u - 1].wait()
                if stage_a:
                    hcp[u - 1].wait()
                if u + 1 <= 7:
                    gcp[u + 1] = pltpu.async_copy(
                        tab.at[sdv.at[2 * (u + 1)]], gbufs[1 - b],
                        gsems[1 - b])
            gcp[u].wait()
            lcp[u].wait()
            _compute(gbufs[b], lbufs[b])
            if stage_a:
                hcp[u] = pltpu.async_copy(
                    gbufs[b], h0.at[pl.ds((base + u) * CH, CH)], hsems[b])
            scp[u] = pltpu.async_copy(gbufs[b], acc.at[sdv.at[2 * u + 1]],
                                      ssems[b], add=True)
            if u + 2 <= 7:
                lcp[u + 2] = pltpu.async_copy(
                    lin.at[pl.ds((base + u + 2) * CH, CH)], lbufs[b], lsems[b])
        scp[7].wait()
        if stage_a:
            hcp[7].wait()
        return 0
    lax.fori_loop(0, nblk, _blk, 0)

    plsc.subcore_barrier()
    for k in range(nz):
        pltpu.sync_copy(acc.at[pl.ds(row0 + k * CH, CH)],
                        part.at[cid, pl.ds(row0 + k * CH, CH)])
    if rem:
        pltpu.sync_copy(acc.at[pl.ds(row0 + nz * CH, rem)],
                        part.at[cid, pl.ds(row0 + nz * CH, rem)])


def _make_edge_pass(stage_a):
    mesh = plsc.VectorSubcoreMesh(core_axis_name="c", subcore_axis_name="s")
    out_type = [jax.ShapeDtypeStruct((NC, NPAD, D_H), _f32)]
    if stage_a:
        out_type += [jax.ShapeDtypeStruct((EPAD, D_H), _f32),
                     jax.ShapeDtypeStruct((BIPAD, D_H), _f32)]
    scratch = [
        pltpu.VMEM((16, CH), jnp.int32),
        pltpu.VMEM((CH, D_H), _f32),
        pltpu.VMEM((CH, D_H), _f32),
        pltpu.VMEM((CH, D_H), _f32),
        pltpu.VMEM((CH, D_H), _f32),
        pltpu.VMEM_SHARED((NPAD, D_H), _f32),
    ] + [pltpu.SemaphoreType.DMA] * 8
    return pl.kernel(
        functools.partial(_edge_pass_body, stage_a),
        out_type=tuple(out_type),
        mesh=mesh,
        scratch_types=tuple(scratch),
    )


_edge_pass_a = _make_edge_pass(True)
_edge_pass_b = _make_edge_pass(False)


def kernel(mol_a_reprs, node_features, edge_features, edge_index, batch_indices,
           W_in, b_in, W_msg, b_msg, W_out, b_out,
           W1, b1, W2, b2, W3, b3, W4, b4):
    src = edge_index[0]
    dst = edge_index[1]
    src2 = jnp.pad(src, (0, EPAD - E)).reshape(EPAD // CH, CH)
    dst2 = jnp.pad(dst, (0, EPAD - E), constant_values=N).reshape(EPAD // CH, CH)
    sd2 = jnp.stack([src2, dst2], axis=1).reshape(2 * EPAD // CH, CH)
    bi2 = jnp.pad(batch_indices, (0, BIPAD - N)).reshape(BIPAD // CH, CH)
    ef8 = jnp.pad(edge_features, ((0, EPAD - E), (0, 0))).reshape(EPAD // 8, 8 * D_EDGE)

    wbd = jnp.zeros((8 * D_EDGE, 8 * D_H), _f32)
    w_bot = W_in[D_FEAT:]
    for j in range(8):
        wbd = wbd.at[j * D_EDGE:(j + 1) * D_EDGE, j * D_H:(j + 1) * D_H].set(w_bot)

    p_tab = pl.pallas_call(
        _table_body,
        out_shape=jax.ShapeDtypeStruct((N, D_H), _f32),
    )(node_features, W_in[:D_FEAT], b_in.reshape(1, D_H))

    RB = 640
    nrb = (EPAD // 8) // RB
    efw8 = pl.pallas_call(
        _efw_body,
        grid=(nrb,),
        in_specs=[
            pl.BlockSpec((RB, 8 * D_EDGE), lambda i: (i, 0)),
            pl.BlockSpec((8 * D_EDGE, 8 * D_H), lambda i: (0, 0)),
        ],
        out_specs=pl.BlockSpec((RB, 8 * D_H), lambda i: (i, 0)),
        out_shape=jax.ShapeDtypeStruct((EPAD // 8, 8 * D_H), _f32),
    )(ef8, wbd)
    efw = efw8.reshape(EPAD, D_H)

    parts, h0, mg = _edge_pass_a(sd2, p_tab, efw, bi2, mol_a_reprs)

    gmm = pl.pallas_call(
        _gmm_body,
        out_shape=jax.ShapeDtypeStruct((NPAD, D_H), _f32),
    )
    b_msg_r = b_msg.reshape(1, D_H)
    for _ in range(STEPS):
        g = gmm(parts[0], parts[1], W_msg, b_msg_r)
        parts = _edge_pass_b(sd2, g, h0)[0]

    FB = 1000
    nfb = N // FB
    row_spec = lambda shp: pl.BlockSpec((FB, shp), lambda i: (i, 0))
    full = lambda r, c: pl.BlockSpec((r, c), lambda i: (0, 0))
    out2d = pl.pallas_call(
        _final_body,
        grid=(nfb,),
        in_specs=[
            row_spec(D_FEAT), row_spec(D_H), row_spec(D_H), row_spec(D_H),
            full(D_FEAT, D_H), full(D_H, D_H), full(1, D_H),
            full(D_H, 256), full(D_H, 256), full(1, 256),
            full(256, 128), full(1, 128),
            full(128, 64), full(1, 64),
            full(1, 64),
            pl.BlockSpec(memory_space=pltpu.SMEM),
        ],
        out_specs=row_spec(D_H),
        out_shape=jax.ShapeDtypeStruct((N, D_H), _f32),
    )(node_features, parts[0], parts[1], mg,
      W_out[:D_FEAT], W_out[D_FEAT:], b_out.reshape(1, D_H),
      W1[:D_H], W1[D_H:], b1.reshape(1, 256),
      W2, b2.reshape(1, 128),
      W3, b3.reshape(1, 64),
      W4[:, 0].reshape(1, 64), b4)
    return out2d[:, 0]
